# baseline (speedup 1.0000x reference)
import jax
import jax.numpy as jnp
from jax import lax
from jax.experimental import pallas as pl
from jax.experimental.pallas import tpu as pltpu
from jax.experimental.pallas import tpu_sc as plsc

DIM = 32
BATCH = 16384
PACK = 4
ROW = PACK * DIM
NC, NS = 2, 16
NW = NC * NS
B_PER_W = BATCH // NW
CHUNK = 128
NCHUNK = B_PER_W // CHUNK
TBLK = 8192
SANITY_EPS = 1e-08


def _tpack_kernel(xp_ref, xb_ref, op_ref, ob_ref):
    op_ref[...] = xp_ref[...].T
    ob_ref[...] = xb_ref[...].T


def _tpack(ept, ebt, interpret=False):
    n = ept.shape[1]
    grid = (pl.cdiv(n, TBLK),)
    in_spec = pl.BlockSpec((DIM, TBLK), lambda i: (0, i))
    out_spec = pl.BlockSpec((TBLK, DIM), lambda i: (i, 0))
    return pl.pallas_call(
        _tpack_kernel,
        grid=grid,
        in_specs=[in_spec, in_spec],
        out_specs=[out_spec, out_spec],
        out_shape=[jax.ShapeDtypeStruct((n, DIM), jnp.float32),
                   jax.ShapeDtypeStruct((n, DIM), jnp.float32)],
        interpret=interpret,
    )(ept, ebt)


def _sc_gather_kernel(ep_hbm, eb_hbm, rp_hbm, h4_hbm, t4_hbm, r_hbm,
                      o_hp, o_tp, o_hb, o_tb, o_rc,
                      ih, it, ir, ebuf0, ebuf1, rbuf0, rbuf1,
                      gs0, gs1, ws0, ws1):
    wid = lax.axis_index("s") * NC + lax.axis_index("c")
    base = wid * B_PER_W
    pltpu.sync_copy(h4_hbm.at[wid], ih)
    pltpu.sync_copy(t4_hbm.at[wid], it)
    pltpu.sync_copy(r_hbm.at[wid], ir)

    bufs = ((ebuf0, ebuf1), (rbuf0, rbuf1))
    gsem = (gs0, gs1)
    wsem = (ws0, ws1)
    jobs = []
    for bi, (tbl, idx, out) in enumerate(
            ((ep_hbm, ih, o_hp), (ep_hbm, it, o_tp),
             (eb_hbm, ih, o_hb), (eb_hbm, it, o_tb),
             (rp_hbm, ir, o_rc))):
        for c in range(NCHUNK):
            jobs.append((tbl, idx, out, c, 1 if bi == 4 else 0))

    g = [None, None]
    w = [None, None]
    prev = None
    for k, (tbl, idx, out, c, which) in enumerate(jobs):
        p = k % 2
        if w[p] is not None:
            w[p].wait()
        g[p] = pltpu.async_copy(tbl.at[idx.at[c]], bufs[which][p], gsem[p])
        if prev is not None:
            pout, pc, pp, pw = prev
            g[pp].wait()
            w[pp] = pltpu.async_copy(
                bufs[pw][pp], pout.at[pl.ds(base + pc * CHUNK, CHUNK)],
                wsem[pp])
        prev = (out, c, p, which)
    pout, pc, pp, pw = prev
    g[pp].wait()
    w[pp] = pltpu.async_copy(
        bufs[pw][pp], pout.at[pl.ds(base + pc * CHUNK, CHUNK)], wsem[pp])
    w[0].wait()
    w[1].wait()


def _sc_gather(ep4, eb4, relp, h4, t4, rr):
    mesh = plsc.VectorSubcoreMesh(core_axis_name="c", subcore_axis_name="s")
    f32 = jnp.float32
    out_type = [
        jax.ShapeDtypeStruct((BATCH, DIM), f32),
        jax.ShapeDtypeStruct((BATCH, DIM), f32),
        jax.ShapeDtypeStruct((BATCH, DIM), f32),
        jax.ShapeDtypeStruct((BATCH, DIM), f32),
        jax.ShapeDtypeStruct((BATCH, ROW), f32),
    ]
    scratch_types = [
        pltpu.VMEM((NCHUNK, CHUNK), jnp.int32),
        pltpu.VMEM((NCHUNK, CHUNK), jnp.int32),
        pltpu.VMEM((NCHUNK, CHUNK), jnp.int32),
        pltpu.VMEM((CHUNK, DIM), f32),
        pltpu.VMEM((CHUNK, DIM), f32),
        pltpu.VMEM((CHUNK, ROW), f32),
        pltpu.VMEM((CHUNK, ROW), f32),
        pltpu.SemaphoreType.DMA,
        pltpu.SemaphoreType.DMA,
        pltpu.SemaphoreType.DMA,
        pltpu.SemaphoreType.DMA,
    ]
    kern = pl.kernel(
        _sc_gather_kernel, out_type=out_type, mesh=mesh,
        scratch_types=scratch_types,
        compiler_params=pltpu.CompilerParams(use_tc_tiling_on_sc=False))
    return kern(ep4, eb4, relp, h4, t4, rr)


def _rel_pack_kernel(rb_ref, rs_ref, sm_ref, o_ref):
    rb = rb_ref[...]
    rs = rs_ref[...]
    sm = sm_ref[...]
    smv = jnp.where(sm > 0, sm, jnp.exp(sm) - 1.0) + 1.0

    def pnorm(x):
        lg = jnp.log(jnp.abs(x) + SANITY_EPS)
        return x / jnp.exp(jnp.mean(lg, axis=1, keepdims=True))

    rd_h = smv[:, 0:1] * pnorm(rs[:, 0:DIM])
    rd_t = smv[:, 1:2] * pnorm(rs[:, DIM:2 * DIM])
    o_ref[...] = jnp.concatenate([rb, rd_h, rd_t], axis=1)


def _rel_pack(rb64, rs64, sm2, interpret=False):
    nrel = rb64.shape[0]
    return pl.pallas_call(
        _rel_pack_kernel,
        out_shape=jax.ShapeDtypeStruct((nrel, ROW), jnp.float32),
        interpret=interpret,
    )(rb64, rs64, sm2)


def _tc_math_kernel(hp_ref, tp_ref, hb_ref, tb_ref, rc_ref, o_ref):
    hp = hp_ref[...]
    tp = tp_ref[...]
    hb = hb_ref[...]
    tb = tb_ref[...]
    rc = rc_ref[...]

    hbn = hb / jnp.maximum(
        jnp.sqrt(jnp.sum(hb * hb, axis=1, keepdims=True)), 1e-12)
    tbn = tb / jnp.maximum(
        jnp.sqrt(jnp.sum(tb * tb, axis=1, keepdims=True)), 1e-12)
    bumped_h = hp + tbn
    bumped_t = tp + hbn

    rb_h = rc[:, 0:DIM]
    rb_t = rc[:, DIM:2 * DIM]
    rd_h = rc[:, 2 * DIM:3 * DIM]
    rd_t = rc[:, 3 * DIM:4 * DIM]

    def box_dist(pt, base, delta):
        w = jnp.abs(delta)
        low = base - 0.5 * w
        high = base + 0.5 * w
        center = 0.5 * (low + high)
        width = high - low
        wp1 = width + 1.0
        inside = jnp.logical_and(pt >= low, pt <= high)
        d_in = jnp.abs(pt - center) / wp1
        d_out = jnp.abs(pt - center) * wp1 - 0.5 * width * (wp1 - 1.0 / wp1)
        return jnp.where(inside, d_in, d_out)

    d_h = box_dist(bumped_h, rb_h, rd_h)
    d_t = box_dist(bumped_t, rb_t, rd_t)
    o_ref[...] = -(jnp.sqrt(jnp.sum(d_h * d_h, axis=1))
                   + jnp.sqrt(jnp.sum(d_t * d_t, axis=1)))


def _tc_math(hp, tp, hb, tb, rc, interpret=False):
    bw = 2048
    grid = (BATCH // bw,)
    ent_spec = pl.BlockSpec((bw, DIM), lambda i: (i, 0))
    return pl.pallas_call(
        _tc_math_kernel,
        grid=grid,
        in_specs=[ent_spec, ent_spec, ent_spec, ent_spec,
                  pl.BlockSpec((bw, ROW), lambda i: (i, 0))],
        out_specs=pl.BlockSpec((bw,), lambda i: (i,)),
        out_shape=jax.ShapeDtypeStruct((BATCH,), jnp.float32),
        interpret=interpret,
    )(hp, tp, hb, tb, rc)


def kernel(entity_points, entity_bumps, rel_bases, rel_shapes, scale_mult,
           heads, tails, rels):
    nrel = rel_bases.shape[0]
    ep4, eb4 = _tpack(entity_points.T, entity_bumps.T)
    relp = _rel_pack(rel_bases.reshape(nrel, 2 * DIM),
                     rel_shapes.reshape(nrel, 2 * DIM),
                     scale_mult.reshape(nrel, 2))
    hh = heads.reshape(NW, NCHUNK, CHUNK)
    tt = tails.reshape(NW, NCHUNK, CHUNK)
    rr = rels.reshape(NW, NCHUNK, CHUNK)
    hp, tp, hb, tb, rc = _sc_gather(ep4, eb4, relp, hh, tt, rr)
    return _tc_math(hp, tp, hb, tb, rc)


# speedup vs baseline: 1.8411x; 1.8411x over previous
import jax
import jax.numpy as jnp
from jax import lax
from jax.experimental import pallas as pl
from jax.experimental.pallas import tpu as pltpu
from jax.experimental.pallas import tpu_sc as plsc

DIM = 32
BATCH = 16384
PACK = 4
ROW = PACK * DIM
NC, NS = 2, 16
NW = NC * NS
B_PER_W = BATCH // NW
CHUNK = 128
NCHUNK = B_PER_W // CHUNK
TBLK = 4096
SANITY_EPS = 1e-08


def _tpack_kernel(xp_ref, xb_ref, op_ref, ob_ref):
    q = TBLK // PACK
    op_ref[...] = jnp.concatenate(
        [xp_ref[:, p * q:(p + 1) * q].T for p in range(PACK)], axis=1)
    ob_ref[...] = jnp.concatenate(
        [xb_ref[:, p * q:(p + 1) * q].T for p in range(PACK)], axis=1)


def _tpack(ept, ebt, interpret=False):
    n = ept.shape[1]
    nblk = pl.cdiv(n, TBLK)
    grid = (nblk,)
    in_spec = pl.BlockSpec((DIM, TBLK), lambda i: (0, i))
    out_spec = pl.BlockSpec((TBLK // PACK, ROW), lambda i: (i, 0))
    n4 = nblk * (TBLK // PACK)
    return pl.pallas_call(
        _tpack_kernel,
        grid=grid,
        in_specs=[in_spec, in_spec],
        out_specs=[out_spec, out_spec],
        out_shape=[jax.ShapeDtypeStruct((n4, ROW), jnp.float32),
                   jax.ShapeDtypeStruct((n4, ROW), jnp.float32)],
        interpret=interpret,
    )(ept, ebt)


def _sc_gather_kernel(ep_hbm, eb_hbm, rp_hbm, h4_hbm, t4_hbm, r_hbm,
                      o_hp, o_tp, o_hb, o_tb, o_rc,
                      ih, it, ir, buf0, buf1,
                      gs0, gs1, ws0, ws1):
    wid = lax.axis_index("s") * NC + lax.axis_index("c")
    base = wid * B_PER_W
    pltpu.sync_copy(h4_hbm.at[wid], ih)
    pltpu.sync_copy(t4_hbm.at[wid], it)
    pltpu.sync_copy(r_hbm.at[wid], ir)

    bufs = (buf0, buf1)
    gsem = (gs0, gs1)
    wsem = (ws0, ws1)
    jobs = []
    for tbl, idx, out in ((ep_hbm, ih, o_hp), (ep_hbm, it, o_tp),
                          (eb_hbm, ih, o_hb), (eb_hbm, it, o_tb),
                          (rp_hbm, ir, o_rc)):
        for c in range(NCHUNK):
            jobs.append((tbl, idx, out, c))

    g = [None, None]
    w = [None, None]
    prev = None
    for k, (tbl, idx, out, c) in enumerate(jobs):
        p = k % 2
        if w[p] is not None:
            w[p].wait()
        g[p] = pltpu.async_copy(tbl.at[idx.at[c]], bufs[p], gsem[p])
        if prev is not None:
            pout, pc, pp = prev
            g[pp].wait()
            w[pp] = pltpu.async_copy(
                bufs[pp], pout.at[pl.ds(base + pc * CHUNK, CHUNK)],
                wsem[pp])
        prev = (out, c, p)
    pout, pc, pp = prev
    g[pp].wait()
    w[pp] = pltpu.async_copy(
        bufs[pp], pout.at[pl.ds(base + pc * CHUNK, CHUNK)], wsem[pp])
    w[0].wait()
    w[1].wait()


def _sc_gather(ep4, eb4, relp, h4, t4, rr):
    mesh = plsc.VectorSubcoreMesh(core_axis_name="c", subcore_axis_name="s")
    f32 = jnp.float32
    out_type = [
        jax.ShapeDtypeStruct((BATCH, ROW), f32),
        jax.ShapeDtypeStruct((BATCH, ROW), f32),
        jax.ShapeDtypeStruct((BATCH, ROW), f32),
        jax.ShapeDtypeStruct((BATCH, ROW), f32),
        jax.ShapeDtypeStruct((BATCH, ROW), f32),
    ]
    scratch_types = [
        pltpu.VMEM((NCHUNK, CHUNK), jnp.int32),
        pltpu.VMEM((NCHUNK, CHUNK), jnp.int32),
        pltpu.VMEM((NCHUNK, CHUNK), jnp.int32),
        pltpu.VMEM((CHUNK, ROW), f32),
        pltpu.VMEM((CHUNK, ROW), f32),
        pltpu.SemaphoreType.DMA,
        pltpu.SemaphoreType.DMA,
        pltpu.SemaphoreType.DMA,
        pltpu.SemaphoreType.DMA,
    ]
    kern = pl.kernel(
        _sc_gather_kernel, out_type=out_type, mesh=mesh,
        scratch_types=scratch_types,
        compiler_params=pltpu.CompilerParams(use_tc_tiling_on_sc=False))
    return kern(ep4, eb4, relp, h4, t4, rr)


def _rel_pack_kernel(rb_ref, rs_ref, sm_ref, o_ref):
    rb = rb_ref[...]
    rs = rs_ref[...]
    sm = sm_ref[...]
    smv = jnp.where(sm > 0, sm, jnp.exp(sm) - 1.0) + 1.0

    def pnorm(x):
        lg = jnp.log(jnp.abs(x) + SANITY_EPS)
        return x / jnp.exp(jnp.mean(lg, axis=1, keepdims=True))

    rd_h = smv[:, 0:1] * pnorm(rs[:, 0:DIM])
    rd_t = smv[:, 1:2] * pnorm(rs[:, DIM:2 * DIM])
    o_ref[...] = jnp.concatenate([rb, rd_h, rd_t], axis=1)


def _rel_pack(rb64, rs64, sm2, interpret=False):
    nrel = rb64.shape[0]
    return pl.pallas_call(
        _rel_pack_kernel,
        out_shape=jax.ShapeDtypeStruct((nrel, ROW), jnp.float32),
        interpret=interpret,
    )(rb64, rs64, sm2)


def _tc_math_kernel(hp_ref, tp_ref, hb_ref, tb_ref, rc_ref,
                    hm_ref, tm_ref, o_ref):
    hm = hm_ref[...]
    tm = tm_ref[...]

    def select(packed, m):
        out = packed[:, 0:DIM] * (m == 0)
        for p in range(1, PACK):
            out = out + packed[:, p * DIM:(p + 1) * DIM] * (m == p)
        return out

    hp = select(hp_ref[...], hm)
    tp = select(tp_ref[...], tm)
    hb = select(hb_ref[...], hm)
    tb = select(tb_ref[...], tm)
    rc = rc_ref[...]

    hbn = hb / jnp.maximum(
        jnp.sqrt(jnp.sum(hb * hb, axis=1, keepdims=True)), 1e-12)
    tbn = tb / jnp.maximum(
        jnp.sqrt(jnp.sum(tb * tb, axis=1, keepdims=True)), 1e-12)
    bumped_h = hp + tbn
    bumped_t = tp + hbn

    rb_h = rc[:, 0:DIM]
    rb_t = rc[:, DIM:2 * DIM]
    rd_h = rc[:, 2 * DIM:3 * DIM]
    rd_t = rc[:, 3 * DIM:4 * DIM]

    def box_dist(pt, base, delta):
        w = jnp.abs(delta)
        low = base - 0.5 * w
        high = base + 0.5 * w
        center = 0.5 * (low + high)
        width = high - low
        wp1 = width + 1.0
        inside = jnp.logical_and(pt >= low, pt <= high)
        d_in = jnp.abs(pt - center) / wp1
        d_out = jnp.abs(pt - center) * wp1 - 0.5 * width * (wp1 - 1.0 / wp1)
        return jnp.where(inside, d_in, d_out)

    d_h = box_dist(bumped_h, rb_h, rd_h)
    d_t = box_dist(bumped_t, rb_t, rd_t)
    o_ref[...] = -(jnp.sqrt(jnp.sum(d_h * d_h, axis=1))
                   + jnp.sqrt(jnp.sum(d_t * d_t, axis=1)))


def _tc_math(hp, tp, hb, tb, rc, hm, tm, interpret=False):
    bw = 2048
    grid = (BATCH // bw,)
    row_spec = pl.BlockSpec((bw, ROW), lambda i: (i, 0))
    m_spec = pl.BlockSpec((bw, 1), lambda i: (i, 0))
    return pl.pallas_call(
        _tc_math_kernel,
        grid=grid,
        in_specs=[row_spec, row_spec, row_spec, row_spec, row_spec,
                  m_spec, m_spec],
        out_specs=pl.BlockSpec((bw,), lambda i: (i,)),
        out_shape=jax.ShapeDtypeStruct((BATCH,), jnp.float32),
        interpret=interpret,
    )(hp, tp, hb, tb, rc, hm, tm)


def kernel(entity_points, entity_bumps, rel_bases, rel_shapes, scale_mult,
           heads, tails, rels):
    nrel = rel_bases.shape[0]
    ep4, eb4 = _tpack(entity_points.T, entity_bumps.T)
    relp = _rel_pack(rel_bases.reshape(nrel, 2 * DIM),
                     rel_shapes.reshape(nrel, 2 * DIM),
                     scale_mult.reshape(nrel, 2))
    q = TBLK // PACK
    hh = ((heads // TBLK) * q + heads % q).reshape(NW, NCHUNK, CHUNK)
    tt = ((tails // TBLK) * q + tails % q).reshape(NW, NCHUNK, CHUNK)
    rr = rels.reshape(NW, NCHUNK, CHUNK)
    hm = ((heads // q) % PACK).reshape(BATCH, 1)
    tm = ((tails // q) % PACK).reshape(BATCH, 1)
    hp, tp, hb, tb, rc = _sc_gather(ep4, eb4, relp, hh, tt, rr)
    return _tc_math(hp, tp, hb, tb, rc, hm, tm)


# speedup vs baseline: 1.8792x; 1.0207x over previous
import jax
import jax.numpy as jnp
from jax import lax
from jax.experimental import pallas as pl
from jax.experimental.pallas import tpu as pltpu
from jax.experimental.pallas import tpu_sc as plsc

DIM = 32
BATCH = 16384
PACK = 4
ROW = PACK * DIM
NC, NS = 2, 16
NW = NC * NS
B_PER_W = BATCH // NW
CHUNK = 128
NCHUNK = B_PER_W // CHUNK
TBLK = 8192
SANITY_EPS = 1e-08


def _tpack_kernel(xp_ref, xb_ref, op_ref, ob_ref):
    q = TBLK // PACK
    op_ref[...] = jnp.concatenate(
        [xp_ref[:, p * q:(p + 1) * q].T for p in range(PACK)], axis=1)
    ob_ref[...] = jnp.concatenate(
        [xb_ref[:, p * q:(p + 1) * q].T for p in range(PACK)], axis=1)


def _tpack(ept, ebt, interpret=False):
    n = ept.shape[1]
    nblk = pl.cdiv(n, TBLK)
    grid = (nblk,)
    in_spec = pl.BlockSpec((DIM, TBLK), lambda i: (0, i))
    out_spec = pl.BlockSpec((TBLK // PACK, ROW), lambda i: (i, 0))
    n4 = nblk * (TBLK // PACK)
    return pl.pallas_call(
        _tpack_kernel,
        grid=grid,
        in_specs=[in_spec, in_spec],
        out_specs=[out_spec, out_spec],
        out_shape=[jax.ShapeDtypeStruct((n4, ROW), jnp.float32),
                   jax.ShapeDtypeStruct((n4, ROW), jnp.float32)],
        interpret=interpret,
    )(ept, ebt)


def _sc_gather_kernel(ep_hbm, eb_hbm, rp_hbm, h4_hbm, t4_hbm, r_hbm,
                      o_hp, o_tp, o_hb, o_tb, o_rc,
                      ih, it, ir, buf0, buf1,
                      gs0, gs1, ws0, ws1):
    wid = lax.axis_index("s") * NC + lax.axis_index("c")
    base = wid * B_PER_W
    pltpu.sync_copy(h4_hbm.at[wid], ih)
    pltpu.sync_copy(t4_hbm.at[wid], it)
    pltpu.sync_copy(r_hbm.at[wid], ir)

    bufs = (buf0, buf1)
    gsem = (gs0, gs1)
    wsem = (ws0, ws1)
    jobs = []
    for tbl, idx, out in ((ep_hbm, ih, o_hp), (ep_hbm, it, o_tp),
                          (eb_hbm, ih, o_hb), (eb_hbm, it, o_tb),
                          (rp_hbm, ir, o_rc)):
        for c in range(NCHUNK):
            jobs.append((tbl, idx, out, c))

    g = [None, None]
    w = [None, None]
    prev = None
    for k, (tbl, idx, out, c) in enumerate(jobs):
        p = k % 2
        if w[p] is not None:
            w[p].wait()
        g[p] = pltpu.async_copy(tbl.at[idx.at[c]], bufs[p], gsem[p])
        if prev is not None:
            pout, pc, pp = prev
            g[pp].wait()
            w[pp] = pltpu.async_copy(
                bufs[pp], pout.at[pl.ds(base + pc * CHUNK, CHUNK)],
                wsem[pp])
        prev = (out, c, p)
    pout, pc, pp = prev
    g[pp].wait()
    w[pp] = pltpu.async_copy(
        bufs[pp], pout.at[pl.ds(base + pc * CHUNK, CHUNK)], wsem[pp])
    w[0].wait()
    w[1].wait()


def _sc_gather(ep4, eb4, relp, h4, t4, rr):
    mesh = plsc.VectorSubcoreMesh(core_axis_name="c", subcore_axis_name="s")
    f32 = jnp.float32
    out_type = [
        jax.ShapeDtypeStruct((BATCH, ROW), f32),
        jax.ShapeDtypeStruct((BATCH, ROW), f32),
        jax.ShapeDtypeStruct((BATCH, ROW), f32),
        jax.ShapeDtypeStruct((BATCH, ROW), f32),
        jax.ShapeDtypeStruct((BATCH, ROW), f32),
    ]
    scratch_types = [
        pltpu.VMEM((NCHUNK, CHUNK), jnp.int32),
        pltpu.VMEM((NCHUNK, CHUNK), jnp.int32),
        pltpu.VMEM((NCHUNK, CHUNK), jnp.int32),
        pltpu.VMEM((CHUNK, ROW), f32),
        pltpu.VMEM((CHUNK, ROW), f32),
        pltpu.SemaphoreType.DMA,
        pltpu.SemaphoreType.DMA,
        pltpu.SemaphoreType.DMA,
        pltpu.SemaphoreType.DMA,
    ]
    kern = pl.kernel(
        _sc_gather_kernel, out_type=out_type, mesh=mesh,
        scratch_types=scratch_types,
        compiler_params=pltpu.CompilerParams(use_tc_tiling_on_sc=False))
    return kern(ep4, eb4, relp, h4, t4, rr)


def _rel_pack_kernel(rb_ref, rs_ref, sm_ref, o_ref):
    rb = rb_ref[...]
    rs = rs_ref[...]
    sm = sm_ref[...]
    smv = jnp.where(sm > 0, sm, jnp.exp(sm) - 1.0) + 1.0

    def pnorm(x):
        lg = jnp.log(jnp.abs(x) + SANITY_EPS)
        return x / jnp.exp(jnp.mean(lg, axis=1, keepdims=True))

    rd_h = smv[:, 0:1] * pnorm(rs[:, 0:DIM])
    rd_t = smv[:, 1:2] * pnorm(rs[:, DIM:2 * DIM])
    o_ref[...] = jnp.concatenate([rb, rd_h, rd_t], axis=1)


def _rel_pack(rb64, rs64, sm2, interpret=False):
    nrel = rb64.shape[0]
    return pl.pallas_call(
        _rel_pack_kernel,
        out_shape=jax.ShapeDtypeStruct((nrel, ROW), jnp.float32),
        interpret=interpret,
    )(rb64, rs64, sm2)


def _tc_math_kernel(hp_ref, tp_ref, hb_ref, tb_ref, rc_ref,
                    hm_ref, tm_ref, o_ref):
    hm = hm_ref[...]
    tm = tm_ref[...]

    def select(packed, m):
        out = packed[:, 0:DIM] * (m == 0)
        for p in range(1, PACK):
            out = out + packed[:, p * DIM:(p + 1) * DIM] * (m == p)
        return out

    hp = select(hp_ref[...], hm)
    tp = select(tp_ref[...], tm)
    hb = select(hb_ref[...], hm)
    tb = select(tb_ref[...], tm)
    rc = rc_ref[...]

    hbn = hb / jnp.maximum(
        jnp.sqrt(jnp.sum(hb * hb, axis=1, keepdims=True)), 1e-12)
    tbn = tb / jnp.maximum(
        jnp.sqrt(jnp.sum(tb * tb, axis=1, keepdims=True)), 1e-12)
    bumped_h = hp + tbn
    bumped_t = tp + hbn

    rb_h = rc[:, 0:DIM]
    rb_t = rc[:, DIM:2 * DIM]
    rd_h = rc[:, 2 * DIM:3 * DIM]
    rd_t = rc[:, 3 * DIM:4 * DIM]

    def box_dist(pt, base, delta):
        w = jnp.abs(delta)
        low = base - 0.5 * w
        high = base + 0.5 * w
        center = 0.5 * (low + high)
        width = high - low
        wp1 = width + 1.0
        inside = jnp.logical_and(pt >= low, pt <= high)
        d_in = jnp.abs(pt - center) / wp1
        d_out = jnp.abs(pt - center) * wp1 - 0.5 * width * (wp1 - 1.0 / wp1)
        return jnp.where(inside, d_in, d_out)

    d_h = box_dist(bumped_h, rb_h, rd_h)
    d_t = box_dist(bumped_t, rb_t, rd_t)
    o_ref[...] = -(jnp.sqrt(jnp.sum(d_h * d_h, axis=1))
                   + jnp.sqrt(jnp.sum(d_t * d_t, axis=1)))


def _tc_math(hp, tp, hb, tb, rc, hm, tm, interpret=False):
    bw = 2048
    grid = (BATCH // bw,)
    row_spec = pl.BlockSpec((bw, ROW), lambda i: (i, 0))
    m_spec = pl.BlockSpec((bw, 1), lambda i: (i, 0))
    return pl.pallas_call(
        _tc_math_kernel,
        grid=grid,
        in_specs=[row_spec, row_spec, row_spec, row_spec, row_spec,
                  m_spec, m_spec],
        out_specs=pl.BlockSpec((bw,), lambda i: (i,)),
        out_shape=jax.ShapeDtypeStruct((BATCH,), jnp.float32),
        interpret=interpret,
    )(hp, tp, hb, tb, rc, hm, tm)


def kernel(entity_points, entity_bumps, rel_bases, rel_shapes, scale_mult,
           heads, tails, rels):
    nrel = rel_bases.shape[0]
    ep4, eb4 = _tpack(entity_points.T, entity_bumps.T)
    relp = _rel_pack(rel_bases.reshape(nrel, 2 * DIM),
                     rel_shapes.reshape(nrel, 2 * DIM),
                     scale_mult.reshape(nrel, 2))
    q = TBLK // PACK
    hh = ((heads // TBLK) * q + heads % q).reshape(NW, NCHUNK, CHUNK)
    tt = ((tails // TBLK) * q + tails % q).reshape(NW, NCHUNK, CHUNK)
    rr = rels.reshape(NW, NCHUNK, CHUNK)
    hm = ((heads // q) % PACK).reshape(BATCH, 1)
    tm = ((tails // q) % PACK).reshape(BATCH, 1)
    hp, tp, hb, tb, rc = _sc_gather(ep4, eb4, relp, hh, tt, rr)
    return _tc_math(hp, tp, hb, tb, rc, hm, tm)
